# baseline (speedup 1.0000x reference)
import functools

import numpy as np
import jax
import jax.numpy as jnp
from jax.experimental import pallas as pl

N_PERCENTILES = 10
MIN_PCT = 5
MAX_PCT = 95

_INT_MIN = np.int32(-2147483648)
_INT_MAX = np.int32(2147483647)


def _percentile_kernel(x_ref, out_ref, *, ranks, weights):
    x = x_ref[0]
    n = x.shape[0]
    bits = jax.lax.bitcast_convert_type(x, jnp.int32)
    s = jnp.where(bits < 0, jnp.bitwise_xor(jnp.bitwise_not(bits), _INT_MIN), bits)

    for i, (k, w) in enumerate(zip(ranks, weights)):
        def body(j, t):
            bit = jnp.left_shift(np.int32(1), 31 - j)
            t_try = jnp.bitwise_or(t, bit)
            t_cmp = jnp.bitwise_xor(t_try, _INT_MIN)
            cnt = jnp.sum((s < t_cmp).astype(jnp.int32), axis=0, keepdims=True)
            return jnp.where(cnt <= k, t_try, t)

        t0 = jnp.zeros((1, x.shape[1]), jnp.int32)
        t = jax.lax.fori_loop(0, 32, body, t0)
        s_k = jnp.bitwise_xor(t, _INT_MIN)

        le = jnp.sum((s <= s_k).astype(jnp.int32), axis=0, keepdims=True)
        gt_min = jnp.min(
            jnp.where(s > s_k, s, _INT_MAX), axis=0, keepdims=True
        )
        s_up = jnp.where(le >= k + 2, s_k, gt_min)

        def to_f32(sv):
            fbits = jnp.where(
                sv >= 0, sv, jnp.bitwise_xor(jnp.bitwise_not(sv), _INT_MIN)
            )
            return jax.lax.bitcast_convert_type(fbits, jnp.float32)

        lo_f = to_f32(s_k)
        up_f = to_f32(s_up)
        out_ref[0, i, :] = (lo_f * (1.0 - w) + up_f * w)[0]


def kernel(x):
    b, n, d = x.shape
    fracs = np.linspace(MIN_PCT / 100.0, MAX_PCT / 100.0, N_PERCENTILES)
    idx_float = fracs * (n - 1)
    idx_lower = np.floor(idx_float).astype(np.int32)
    w_upper = (idx_float - idx_lower).astype(np.float32)

    dblk = 128
    grid = (b, d // dblk)
    val = pl.pallas_call(
        functools.partial(
            _percentile_kernel,
            ranks=[int(k) for k in idx_lower],
            weights=[float(w) for w in w_upper],
        ),
        grid=grid,
        in_specs=[pl.BlockSpec((1, n, dblk), lambda bi, di: (bi, 0, di))],
        out_specs=pl.BlockSpec((1, N_PERCENTILES, dblk), lambda bi, di: (bi, 0, di)),
        out_shape=jax.ShapeDtypeStruct((b, N_PERCENTILES, d), jnp.float32),
    )(x)
    return jnp.transpose(val, (0, 2, 1)).reshape(b, d * N_PERCENTILES)


# speedup vs baseline: 2.9522x; 2.9522x over previous
import functools

import numpy as np
import jax
import jax.numpy as jnp
from jax.experimental import pallas as pl

N_PERCENTILES = 10
MIN_PCT = 5
MAX_PCT = 95
SEARCH_BITS = 20

_INT_MIN = np.int32(-2147483648)
_INT_MAX = np.int32(2147483647)


def _tree_sum(m):
    r = m
    while r.shape[0] > 8:
        h = r.shape[0] // 2
        r = r[:h] + r[h:]
    return jnp.sum(r, axis=0, keepdims=True)


def _percentile_kernel(x_ref, out_ref, *, ranks, weights):
    x = x_ref[0]
    dblk = x.shape[1]
    bits = jax.lax.bitcast_convert_type(x, jnp.int32)
    s = jnp.where(bits < 0, jnp.bitwise_xor(jnp.bitwise_not(bits), _INT_MIN), bits)

    nr = len(ranks)
    ks = [np.int32(k) for k in ranks]

    def body(j, ts):
        bit = jnp.left_shift(np.int32(1), 31 - j)
        new = []
        for i in range(nr):
            t = ts[i]
            t_try = jnp.bitwise_or(t, bit)
            t_cmp = jnp.bitwise_xor(t_try, _INT_MIN)
            cnt = _tree_sum((s < t_cmp).astype(jnp.int32))
            new.append(jnp.where(cnt <= ks[i], t_try, t))
        return tuple(new)

    ts0 = tuple(jnp.zeros((1, dblk), jnp.int32) for _ in range(nr))
    ts = jax.lax.fori_loop(0, SEARCH_BITS, body, ts0, unroll=True)

    def to_f32(sv):
        fbits = jnp.where(sv >= 0, sv, jnp.bitwise_xor(jnp.bitwise_not(sv), _INT_MIN))
        return jax.lax.bitcast_convert_type(fbits, jnp.float32)

    for i in range(nr):
        t = ts[i]
        s_k = jnp.bitwise_xor(t, _INT_MIN)
        le = _tree_sum((s <= s_k).astype(jnp.int32))
        gt_min = jnp.min(jnp.where(s > s_k, s, _INT_MAX), axis=0, keepdims=True)
        s_up = jnp.where(le >= ks[i] + 2, s_k, gt_min)

        lo_f = to_f32(s_k)
        up_f = to_f32(s_up)
        w = weights[i]
        out_ref[0, i, :] = (lo_f * (1.0 - w) + up_f * w)[0]


def kernel(x):
    b, n, d = x.shape
    fracs = np.linspace(MIN_PCT / 100.0, MAX_PCT / 100.0, N_PERCENTILES)
    idx_float = fracs * (n - 1)
    idx_lower = np.floor(idx_float).astype(np.int32)
    w_upper = (idx_float - idx_lower).astype(np.float32)

    dblk = 128
    grid = (b, d // dblk)
    val = pl.pallas_call(
        functools.partial(
            _percentile_kernel,
            ranks=[int(k) for k in idx_lower],
            weights=[float(w) for w in w_upper],
        ),
        grid=grid,
        in_specs=[pl.BlockSpec((1, n, dblk), lambda bi, di: (bi, 0, di))],
        out_specs=pl.BlockSpec((1, N_PERCENTILES, dblk), lambda bi, di: (bi, 0, di)),
        out_shape=jax.ShapeDtypeStruct((b, N_PERCENTILES, d), jnp.float32),
    )(x)
    return jnp.transpose(val, (0, 2, 1)).reshape(b, d * N_PERCENTILES)


# speedup vs baseline: 7.4284x; 2.5162x over previous
import functools

import numpy as np
import jax
import jax.numpy as jnp
from jax.experimental import pallas as pl

N_PERCENTILES = 10
MIN_PCT = 5
MAX_PCT = 95

_I16_MIN = np.int16(-32768)
_I16_MAX = np.int16(32767)


def _tree_sum_i16(m):
    r = m
    while r.shape[0] > 1:
        h = r.shape[0] // 2
        r = r[:h] + r[h:]
    return r


def _tree_min_bf16(m):
    r = m
    while r.shape[0] > 1:
        h = r.shape[0] // 2
        r = jnp.minimum(r[:h], r[h:])
    return r


def _percentile_kernel(x_ref, out_ref, *, ranks, weights):
    x = x_ref[0]
    xb = x.astype(jnp.bfloat16)
    bits = jax.lax.bitcast_convert_type(xb, jnp.int16)
    s = jnp.where(
        bits < jnp.int16(0),
        jnp.bitwise_xor(jnp.bitwise_not(bits), _I16_MIN),
        bits,
    )

    nr = len(ranks)
    ks = [np.int16(k) for k in ranks]

    ts = [jnp.zeros((1, x.shape[1]), jnp.int16) for _ in range(nr)]
    for j in range(16):
        bit = np.int16(np.uint16(1 << (15 - j)))
        for i in range(nr):
            t = ts[i]
            t_try = jnp.bitwise_or(t, bit)
            t_cmp = jnp.bitwise_xor(t_try, _I16_MIN)
            cnt = _tree_sum_i16((s < t_cmp).astype(jnp.int16))
            ts[i] = jnp.where(cnt <= ks[i], t_try, t)

    def to_bf16(sv):
        fbits = jnp.where(
            sv >= jnp.int16(0),
            sv,
            jnp.bitwise_xor(jnp.bitwise_not(sv), _I16_MIN),
        )
        return jax.lax.bitcast_convert_type(fbits, jnp.bfloat16)

    big = jnp.asarray(np.float32(3.0e38), jnp.bfloat16)
    for i in range(nr):
        s_k = jnp.bitwise_xor(ts[i], _I16_MIN)
        lo_b = to_bf16(s_k)
        le = _tree_sum_i16((s <= s_k).astype(jnp.int16))
        gt_min = _tree_min_bf16(jnp.where(xb > lo_b, xb, big))
        up_b = jnp.where(le >= ks[i] + jnp.int16(2), lo_b, gt_min)

        lo_f = lo_b.astype(jnp.float32)
        up_f = up_b.astype(jnp.float32)
        w = weights[i]
        out_ref[0, i, :] = (lo_f * (1.0 - w) + up_f * w)[0]


def kernel(x):
    b, n, d = x.shape
    fracs = np.linspace(MIN_PCT / 100.0, MAX_PCT / 100.0, N_PERCENTILES)
    idx_float = fracs * (n - 1)
    idx_lower = np.floor(idx_float).astype(np.int32)
    w_upper = (idx_float - idx_lower).astype(np.float32)

    dblk = 128
    grid = (b, d // dblk)
    val = pl.pallas_call(
        functools.partial(
            _percentile_kernel,
            ranks=[int(k) for k in idx_lower],
            weights=[float(w) for w in w_upper],
        ),
        grid=grid,
        in_specs=[pl.BlockSpec((1, n, dblk), lambda bi, di: (bi, 0, di))],
        out_specs=pl.BlockSpec((1, N_PERCENTILES, dblk), lambda bi, di: (bi, 0, di)),
        out_shape=jax.ShapeDtypeStruct((b, N_PERCENTILES, d), jnp.float32),
    )(x)
    return jnp.transpose(val, (0, 2, 1)).reshape(b, d * N_PERCENTILES)


# speedup vs baseline: 8.9483x; 1.2046x over previous
import functools

import numpy as np
import jax
import jax.numpy as jnp
from jax.experimental import pallas as pl

N_PERCENTILES = 10
MIN_PCT = 5
MAX_PCT = 95
SHARED_LEVELS = 4


def _tree_min_bf16(m):
    r = m
    while r.shape[0] > 1:
        h = r.shape[0] // 2
        r = jnp.minimum(r[:h], r[h:])
    return r


def _percentile_kernel(x_ref, out_ref, *, ranks, weights):
    x = x_ref[0]
    n = x.shape[0]
    xb = x.astype(jnp.bfloat16)
    bits = jax.lax.bitcast_convert_type(xb, jnp.int16)
    s = jnp.where(
        bits < jnp.int16(0),
        jnp.bitwise_xor(jnp.bitwise_not(bits), jnp.int16(-32768)),
        bits,
    )

    nr = len(ranks)
    kf = [np.float32(k) for k in ranks]

    one_b = jnp.asarray(1.0, jnp.bfloat16)
    zero_b = jnp.asarray(0.0, jnp.bfloat16)
    ones_row = jnp.full((1, n), 1.0, jnp.bfloat16)
    dn = (((1,), (0,)), ((), ()))

    def count(mask):
        mb = jnp.where(mask, one_b, zero_b)
        return jax.lax.dot_general(
            ones_row, mb, dn, preferred_element_type=jnp.float32
        )

    def cnt_lt_pattern(tp32):
        t_cmp = (tp32 - np.int32(32768)).astype(jnp.int16)
        return count(s < t_cmp)

    ts = [jnp.zeros((1, x.shape[1]), jnp.int32) for _ in range(nr)]
    for lev in range(16):
        bit = np.int32(1 << (15 - lev))
        if lev < SHARED_LEVELS:
            cnts = []
            for m in range(1 << lev):
                pat = np.int32((2 * m + 1) << (15 - lev))
                cnts.append(count(s < jnp.int16(pat - 32768)))
            for i in range(nr):
                c = cnts[0]
                for m in range(1, 1 << lev):
                    prefix = np.int32(m << (16 - lev))
                    c = jnp.where(ts[i] == prefix, cnts[m], c)
                t_try = jnp.bitwise_or(ts[i], bit)
                ts[i] = jnp.where(c <= kf[i], t_try, ts[i])
        else:
            for i in range(nr):
                t_try = jnp.bitwise_or(ts[i], bit)
                c = cnt_lt_pattern(t_try)
                ts[i] = jnp.where(c <= kf[i], t_try, ts[i])

    def pattern_to_f32(tp32):
        sv = tp32 - np.int32(32768)
        fb = jnp.where(
            sv >= 0,
            sv,
            jnp.bitwise_xor(jnp.bitwise_not(sv), np.int32(-32768)),
        )
        return jax.lax.bitcast_convert_type(
            fb.astype(jnp.int16), jnp.bfloat16
        ).astype(jnp.float32)

    big = jnp.asarray(np.float32(3.0e38), jnp.bfloat16)
    for i in range(nr):
        s_k = (ts[i] - np.int32(32768)).astype(jnp.int16)
        lo_f = pattern_to_f32(ts[i])
        le = count(s <= s_k)
        lo_b = jax.lax.bitcast_convert_type(
            jnp.where(
                s_k >= jnp.int16(0),
                s_k,
                jnp.bitwise_xor(jnp.bitwise_not(s_k), jnp.int16(-32768)),
            ),
            jnp.bfloat16,
        )
        gt_min = _tree_min_bf16(jnp.where(xb > lo_b, xb, big)).astype(jnp.float32)
        up_f = jnp.where(le >= kf[i] + np.float32(2.0), lo_f, gt_min)

        w = weights[i]
        out_ref[0, i, :] = (lo_f * (1.0 - w) + up_f * w)[0]


def kernel(x):
    b, n, d = x.shape
    fracs = np.linspace(MIN_PCT / 100.0, MAX_PCT / 100.0, N_PERCENTILES)
    idx_float = fracs * (n - 1)
    idx_lower = np.floor(idx_float).astype(np.int32)
    w_upper = (idx_float - idx_lower).astype(np.float32)

    dblk = 128
    grid = (b, d // dblk)
    val = pl.pallas_call(
        functools.partial(
            _percentile_kernel,
            ranks=[int(k) for k in idx_lower],
            weights=[float(w) for w in w_upper],
        ),
        grid=grid,
        in_specs=[pl.BlockSpec((1, n, dblk), lambda bi, di: (bi, 0, di))],
        out_specs=pl.BlockSpec((1, N_PERCENTILES, dblk), lambda bi, di: (bi, 0, di)),
        out_shape=jax.ShapeDtypeStruct((b, N_PERCENTILES, d), jnp.float32),
    )(x)
    return jnp.transpose(val, (0, 2, 1)).reshape(b, d * N_PERCENTILES)
